# Initial kernel scaffold; baseline (speedup 1.0000x reference)
import jax
import jax.numpy as jnp
from jax.experimental import pallas as pl


def kernel(x, edge_index, ln1_g, ln1_b, Ws1, Wb1, bb1, gb1, ln2_g, ln2_b, Ws2, Wb2, bb2, gb2):
    raise NotImplementedError("write your pallas kernel here")


# speedup vs baseline: 14.9915x; 14.9915x over previous
import functools

import jax
import jax.numpy as jnp
import numpy as np
from jax import lax
from jax.experimental import pallas as pl
from jax.experimental.pallas import tpu as pltpu
from jax.experimental.pallas import tpu_sc as plsc

N = 10000
D = 128
E = 320000
H = 128
C = 40
G = 4

NPAD = 10240
NTILE = 16
NCORE = 2
W = 128
E_PAD = 323584
EP_W = E_PAD // (NCORE * NTILE)
NWIN = EP_W // W
ROWS_T = NPAD // NTILE

GRID = tuple(np.linspace(-2.0, 2.0, G).tolist())
INV = (G - 1) / 4.0


def _vmesh():
    return plsc.VectorSubcoreMesh(core_axis_name="c", subcore_axis_name="s")


def _sc_degree(dstp, zeros_t, ones_w):
    @functools.partial(
        pl.kernel,
        out_type=jax.ShapeDtypeStruct((NCORE, NPAD), jnp.float32),
        mesh=_vmesh(),
        scratch_types=[
            pltpu.VMEM((1, W), jnp.int32),
            pltpu.VMEM((W,), jnp.float32),
            pltpu.VMEM_SHARED((NPAD,), jnp.float32),
        ],
    )
    def k(d_hbm, z_hbm, one_hbm, out_hbm, didx, ones_v, acc):
        c = lax.axis_index("c")
        s = lax.axis_index("s")
        t0 = s * ROWS_T
        pltpu.sync_copy(z_hbm, acc.at[pl.ds(t0, ROWS_T)])
        pltpu.sync_copy(one_hbm, ones_v)
        plsc.subcore_barrier()
        base = c * (E_PAD // 2) + s * EP_W

        @pl.loop(0, NWIN)
        def _(w):
            off = base + w * W
            pltpu.sync_copy(d_hbm.at[pl.ds(off, W)], didx.at[0])
            pltpu.sync_copy(ones_v, acc.at[didx.at[0]], add=True)

        plsc.subcore_barrier()
        pltpu.sync_copy(acc.at[pl.ds(t0, ROWS_T)], out_hbm.at[c].at[pl.ds(t0, ROWS_T)])

    return k(dstp, zeros_t, ones_w)


def _sc_scatter(y, srcp, dstp, zeros_t, width):
    @functools.partial(
        pl.kernel,
        out_type=jax.ShapeDtypeStruct((NCORE, NPAD, width), jnp.float32),
        mesh=_vmesh(),
        compiler_params=pltpu.CompilerParams(use_tc_tiling_on_sc=False),
        scratch_types=[
            pltpu.VMEM((1, W), jnp.int32),
            pltpu.VMEM((1, W), jnp.int32),
            pltpu.VMEM((W, width), jnp.float32),
            pltpu.VMEM_SHARED((NPAD, width), jnp.float32),
        ],
    )
    def k(y_hbm, s_hbm, d_hbm, z_hbm, out_hbm, sidx, didx, rows, acc):
        c = lax.axis_index("c")
        s = lax.axis_index("s")
        t0 = s * ROWS_T
        pltpu.sync_copy(z_hbm, acc.at[pl.ds(t0, ROWS_T)])
        plsc.subcore_barrier()
        base = c * (E_PAD // 2) + s * EP_W

        @pl.loop(0, NWIN)
        def _(w):
            off = base + w * W
            pltpu.sync_copy(s_hbm.at[pl.ds(off, W)], sidx.at[0])
            pltpu.sync_copy(d_hbm.at[pl.ds(off, W)], didx.at[0])
            pltpu.sync_copy(y_hbm.at[sidx.at[0]], rows)
            pltpu.sync_copy(rows, acc.at[didx.at[0]], add=True)

        plsc.subcore_barrier()
        pltpu.sync_copy(acc.at[pl.ds(t0, ROWS_T)],
                        out_hbm.at[c].at[pl.ds(t0, ROWS_T)])

    return k(y, srcp, dstp, zeros_t)


BR = 400


def _dis_block(dps):
    return lax.rsqrt(dps[0] + dps[1] + 1.0)


def _fkan_body(xin, g, b, Wsg, Wb, bb, nfeat, width):
    m = jnp.mean(xin, axis=1, keepdims=True)
    xc = xin - m
    v = jnp.mean(xc * xc, axis=1, keepdims=True)
    h = xc * lax.rsqrt(v + 1e-5) * g + b
    acc = jnp.dot(xin * jax.nn.sigmoid(xin), Wb,
                  preferred_element_type=jnp.float32)
    for gg in range(G):
        basis = jnp.exp(-(((h - GRID[gg]) * INV) ** 2))
        acc = acc + jnp.dot(basis, Wsg[gg], preferred_element_type=jnp.float32)
    return acc + bb


def _tc_fkan1(x, dps, g, b, Wsg, Wb, bb):
    def body(x_ref, dps_ref, g_ref, b_ref, Wsg_ref, Wb_ref, bb_ref, o_ref):
        dis = _dis_block(dps_ref)
        y = _fkan_body(x_ref[...], g_ref[...], b_ref[...], Wsg_ref, Wb_ref[...],
                       bb_ref[...], D, H)
        o_ref[...] = dis * y

    return pl.pallas_call(
        body,
        grid=(N // BR,),
        in_specs=[
            pl.BlockSpec((BR, D), lambda i: (i, 0)),
            pl.BlockSpec((2, BR, 1), lambda i: (0, i, 0)),
            pl.BlockSpec((1, D), lambda i: (0, 0)),
            pl.BlockSpec((1, D), lambda i: (0, 0)),
            pl.BlockSpec((G, D, H), lambda i: (0, 0, 0)),
            pl.BlockSpec((D, H), lambda i: (0, 0)),
            pl.BlockSpec((1, H), lambda i: (0, 0)),
        ],
        out_specs=pl.BlockSpec((BR, H), lambda i: (i, 0)),
        out_shape=jax.ShapeDtypeStruct((N, H), jnp.float32),
    )(x, dps, g, b, Wsg, Wb, bb)


def _tc_fkan2(x, p, y1, dps, gb1, g2, b2, Wsg2, Wb2, bb2):
    D2 = D + H

    def body(x_ref, p_ref, y1_ref, dps_ref, gb1_ref, g_ref, b_ref, Wsg_ref,
             Wb_ref, bb_ref, o_ref):
        dis = _dis_block(dps_ref)
        h1 = dis * (p_ref[0] + p_ref[1] + y1_ref[...]) + gb1_ref[...]
        hcat = jnp.concatenate([x_ref[...], h1], axis=1)
        y = _fkan_body(hcat, g_ref[...], b_ref[...], Wsg_ref, Wb_ref[...],
                       bb_ref[...], D2, C)
        o_ref[...] = dis * y

    return pl.pallas_call(
        body,
        grid=(N // BR,),
        in_specs=[
            pl.BlockSpec((BR, D), lambda i: (i, 0)),
            pl.BlockSpec((2, BR, H), lambda i: (0, i, 0)),
            pl.BlockSpec((BR, H), lambda i: (i, 0)),
            pl.BlockSpec((2, BR, 1), lambda i: (0, i, 0)),
            pl.BlockSpec((1, H), lambda i: (0, 0)),
            pl.BlockSpec((1, D2), lambda i: (0, 0)),
            pl.BlockSpec((1, D2), lambda i: (0, 0)),
            pl.BlockSpec((G, D2, C), lambda i: (0, 0, 0)),
            pl.BlockSpec((D2, C), lambda i: (0, 0)),
            pl.BlockSpec((1, C), lambda i: (0, 0)),
        ],
        out_specs=pl.BlockSpec((BR, C), lambda i: (i, 0)),
        out_shape=jax.ShapeDtypeStruct((N, C), jnp.float32),
    )(x, p, y1, dps, gb1, g2, b2, Wsg2, Wb2, bb2)


def _tc_final(q, y2, dps, gb2):
    def body(q_ref, y2_ref, dps_ref, gb2_ref, o_ref):
        dis = _dis_block(dps_ref)
        o_ref[...] = dis * (q_ref[0] + q_ref[1] + y2_ref[...]) + gb2_ref[...]

    return pl.pallas_call(
        body,
        grid=(N // BR,),
        in_specs=[
            pl.BlockSpec((2, BR, C), lambda i: (0, i, 0)),
            pl.BlockSpec((BR, C), lambda i: (i, 0)),
            pl.BlockSpec((2, BR, 1), lambda i: (0, i, 0)),
            pl.BlockSpec((1, C), lambda i: (0, 0)),
        ],
        out_specs=pl.BlockSpec((BR, C), lambda i: (i, 0)),
        out_shape=jax.ShapeDtypeStruct((N, C), jnp.float32),
    )(q, y2, dps, gb2)


def kernel(x, edge_index, ln1_g, ln1_b, Ws1, Wb1, bb1, gb1,
           ln2_g, ln2_b, Ws2, Wb2, bb2, gb2):
    src = edge_index[0]
    dst = edge_index[1]
    npads = E_PAD - E
    pad_i = jnp.arange(npads, dtype=jnp.int32)
    srcp = jnp.concatenate([src, pad_i % N])
    dstp = jnp.concatenate([dst, N + pad_i % (NPAD - N)])

    zeros_deg = jnp.zeros((ROWS_T,), jnp.float32)
    ones_w = jnp.ones((W,), jnp.float32)
    dp = _sc_degree(dstp, zeros_deg, ones_w)
    dps = dp[:, :N, None]

    Wsg1 = Ws1.reshape(D, G, H).transpose(1, 0, 2)
    y1 = _tc_fkan1(x, dps, ln1_g[None], ln1_b[None], Wsg1, Wb1, bb1[None])

    zeros_h = jnp.zeros((ROWS_T, H), jnp.float32)
    p = _sc_scatter(y1, srcp, dstp, zeros_h, H)

    D2 = D + H
    Wsg2 = Ws2.reshape(D2, G, C).transpose(1, 0, 2)
    y2 = _tc_fkan2(x, p[:, :N], y1, dps, gb1[None], ln2_g[None], ln2_b[None],
                   Wsg2, Wb2, bb2[None])

    zeros_c = jnp.zeros((ROWS_T, C), jnp.float32)
    q = _sc_scatter(y2, srcp, dstp, zeros_c, C)

    return _tc_final(q[:, :N], y2, dps, gb2[None])


# speedup vs baseline: 22.6164x; 1.5086x over previous
import functools

import jax
import jax.numpy as jnp
import numpy as np
from jax import lax
from jax.experimental import pallas as pl
from jax.experimental.pallas import tpu as pltpu
from jax.experimental.pallas import tpu_sc as plsc

N = 10000
D = 128
E = 320000
H = 128
C = 40
G = 4

NPAD = 10240
NTILE = 16
NCORE = 2
W = 128
E_PAD = 327680
EP_W = E_PAD // (NCORE * NTILE)
NWIN = EP_W // W
ROWS_T = NPAD // NTILE
DEG_K = 8

GRID = tuple(np.linspace(-2.0, 2.0, G).tolist())
INV = (G - 1) / 4.0


def _vmesh():
    return plsc.VectorSubcoreMesh(core_axis_name="c", subcore_axis_name="s")


def _sc_degree(dstp2, zeros_t, ones_w):
    @functools.partial(
        pl.kernel,
        out_type=jax.ShapeDtypeStruct((NCORE, NPAD), jnp.float32),
        mesh=_vmesh(),
        scratch_types=[
            pltpu.VMEM((NWIN, W), jnp.int32),
            pltpu.VMEM((W,), jnp.float32),
            pltpu.VMEM_SHARED((NPAD,), jnp.float32),
            pltpu.SemaphoreType.DMA,
        ],
    )
    def k(d_hbm, z_hbm, one_hbm, out_hbm, didx, ones_v, acc, sem):
        c = lax.axis_index("c")
        s = lax.axis_index("s")
        t0 = s * ROWS_T
        wb = (c * NTILE + s) * NWIN
        pltpu.sync_copy(d_hbm.at[pl.ds(wb, NWIN)], didx)
        pltpu.sync_copy(z_hbm, acc.at[pl.ds(t0, ROWS_T)])
        pltpu.sync_copy(one_hbm, ones_v)
        plsc.subcore_barrier()

        @pl.loop(0, NWIN // DEG_K)
        def _(kk):
            descs = [
                pltpu.async_copy(ones_v, acc.at[didx.at[kk * DEG_K + j]],
                                 sem, add=True)
                for j in range(DEG_K)
            ]
            for dsc in descs:
                dsc.wait()

        plsc.subcore_barrier()
        pltpu.sync_copy(acc.at[pl.ds(t0, ROWS_T)],
                        out_hbm.at[c].at[pl.ds(t0, ROWS_T)])

    return k(dstp2, zeros_t, ones_w)


def _sc_scatter(y, srcp2, dstp2, zeros_t, width, col_split):
    nw = E_PAD // W // (NTILE if col_split else NTILE * NCORE)

    @functools.partial(
        pl.kernel,
        out_type=jax.ShapeDtypeStruct((NCORE, NPAD, width), jnp.float32),
        mesh=_vmesh(),
        compiler_params=pltpu.CompilerParams(use_tc_tiling_on_sc=False),
        scratch_types=[
            pltpu.VMEM((nw, W), jnp.int32),
            pltpu.VMEM((nw, W), jnp.int32),
            pltpu.VMEM((2, W, width), jnp.float32),
            pltpu.VMEM_SHARED((NPAD, width), jnp.float32),
            pltpu.SemaphoreType.DMA,
            pltpu.SemaphoreType.DMA,
            pltpu.SemaphoreType.DMA,
            pltpu.SemaphoreType.DMA,
        ],
    )
    def k(y_hbm, s_hbm, d_hbm, z_hbm, out_hbm, sidx, didx, rows, acc,
          sg0, sg1, ss0, ss1):
        c = lax.axis_index("c")
        s = lax.axis_index("s")
        t0 = s * ROWS_T
        if col_split:
            yref = y_hbm.at[c]
            wb = s * nw
        else:
            yref = y_hbm
            wb = (c * NTILE + s) * nw
        pltpu.sync_copy(s_hbm.at[pl.ds(wb, nw)], sidx)
        pltpu.sync_copy(d_hbm.at[pl.ds(wb, nw)], didx)
        pltpu.sync_copy(z_hbm, acc.at[pl.ds(t0, ROWS_T)])
        plsc.subcore_barrier()

        def g_start(w, b, sem):
            return pltpu.async_copy(yref.at[sidx.at[w]], rows.at[b], sem)

        def g_wait(b, sem):
            pltpu.make_async_copy(yref.at[sidx.at[0]], rows.at[b], sem).wait()

        def s_start(w, b, sem):
            return pltpu.async_copy(rows.at[b], acc.at[didx.at[w]], sem,
                                    add=True)

        g_start(0, 0, sg0)
        g_start(1, 1, sg1)

        @pl.loop(0, nw // 2 - 1)
        def _(kk):
            w0 = kk * 2
            g_wait(0, sg0)
            sd0 = s_start(w0, 0, ss0)
            g_wait(1, sg1)
            sd1 = s_start(w0 + 1, 1, ss1)
            sd0.wait()
            g_start(w0 + 2, 0, sg0)
            sd1.wait()
            g_start(w0 + 3, 1, sg1)

        g_wait(0, sg0)
        sd0 = s_start(nw - 2, 0, ss0)
        g_wait(1, sg1)
        sd1 = s_start(nw - 1, 1, ss1)
        sd0.wait()
        sd1.wait()

        plsc.subcore_barrier()
        pltpu.sync_copy(acc.at[pl.ds(t0, ROWS_T)],
                        out_hbm.at[c].at[pl.ds(t0, ROWS_T)])

    return k(y, srcp2, dstp2, zeros_t)


BR = 400


def _dis_block(dps):
    return lax.rsqrt(dps[0] + dps[1] + 1.0)


def _fkan_body(xin, g, b, Wsg, Wb, bb):
    m = jnp.mean(xin, axis=1, keepdims=True)
    xc = xin - m
    v = jnp.mean(xc * xc, axis=1, keepdims=True)
    h = xc * lax.rsqrt(v + 1e-5) * g + b
    acc = jnp.dot(xin * jax.nn.sigmoid(xin), Wb,
                  preferred_element_type=jnp.float32)
    for gg in range(G):
        basis = jnp.exp(-(((h - GRID[gg]) * INV) ** 2))
        acc = acc + jnp.dot(basis, Wsg[gg], preferred_element_type=jnp.float32)
    return acc + bb


def _tc_fkan1(x, dps, g, b, Wsg, Wb, bb):
    def body(x_ref, dps_ref, g_ref, b_ref, Wsg_ref, Wb_ref, bb_ref, o_ref):
        dis = _dis_block(dps_ref)
        y = dis * _fkan_body(x_ref[...], g_ref[...], b_ref[...], Wsg_ref,
                             Wb_ref[...], bb_ref[...])
        o_ref[0] = y[:, :H // 2]
        o_ref[1] = y[:, H // 2:]

    return pl.pallas_call(
        body,
        grid=(N // BR,),
        in_specs=[
            pl.BlockSpec((BR, D), lambda i: (i, 0)),
            pl.BlockSpec((2, BR, 1), lambda i: (0, i, 0)),
            pl.BlockSpec((1, D), lambda i: (0, 0)),
            pl.BlockSpec((1, D), lambda i: (0, 0)),
            pl.BlockSpec((G, D, H), lambda i: (0, 0, 0)),
            pl.BlockSpec((D, H), lambda i: (0, 0)),
            pl.BlockSpec((1, H), lambda i: (0, 0)),
        ],
        out_specs=pl.BlockSpec((2, BR, H // 2), lambda i: (0, i, 0)),
        out_shape=jax.ShapeDtypeStruct((2, N, H // 2), jnp.float32),
    )(x, dps, g, b, Wsg, Wb, bb)


def _tc_fkan2(x, p, y1, dps, gb1, g2, b2, Wsg2, Wb2, bb2):
    D2 = D + H

    def body(x_ref, p_ref, y1_ref, dps_ref, gb1_ref, g_ref, b_ref, Wsg_ref,
             Wb_ref, bb_ref, o_ref):
        dis = _dis_block(dps_ref)
        hh = p_ref[...] + y1_ref[...]
        h1 = dis * jnp.concatenate([hh[0], hh[1]], axis=1) + gb1_ref[...]
        hcat = jnp.concatenate([x_ref[...], h1], axis=1)
        y = _fkan_body(hcat, g_ref[...], b_ref[...], Wsg_ref, Wb_ref[...],
                       bb_ref[...])
        o_ref[...] = dis * y

    return pl.pallas_call(
        body,
        grid=(N // BR,),
        in_specs=[
            pl.BlockSpec((BR, D), lambda i: (i, 0)),
            pl.BlockSpec((2, BR, H // 2), lambda i: (0, i, 0)),
            pl.BlockSpec((2, BR, H // 2), lambda i: (0, i, 0)),
            pl.BlockSpec((2, BR, 1), lambda i: (0, i, 0)),
            pl.BlockSpec((1, H), lambda i: (0, 0)),
            pl.BlockSpec((1, D2), lambda i: (0, 0)),
            pl.BlockSpec((1, D2), lambda i: (0, 0)),
            pl.BlockSpec((G, D2, C), lambda i: (0, 0, 0)),
            pl.BlockSpec((D2, C), lambda i: (0, 0)),
            pl.BlockSpec((1, C), lambda i: (0, 0)),
        ],
        out_specs=pl.BlockSpec((BR, C), lambda i: (i, 0)),
        out_shape=jax.ShapeDtypeStruct((N, C), jnp.float32),
    )(x, p, y1, dps, gb1, g2, b2, Wsg2, Wb2, bb2)


def _tc_final(q, y2, dps, gb2):
    def body(q_ref, y2_ref, dps_ref, gb2_ref, o_ref):
        dis = _dis_block(dps_ref)
        o_ref[...] = dis * (q_ref[0] + q_ref[1] + y2_ref[...]) + gb2_ref[...]

    return pl.pallas_call(
        body,
        grid=(N // BR,),
        in_specs=[
            pl.BlockSpec((2, BR, C), lambda i: (0, i, 0)),
            pl.BlockSpec((BR, C), lambda i: (i, 0)),
            pl.BlockSpec((2, BR, 1), lambda i: (0, i, 0)),
            pl.BlockSpec((1, C), lambda i: (0, 0)),
        ],
        out_specs=pl.BlockSpec((BR, C), lambda i: (i, 0)),
        out_shape=jax.ShapeDtypeStruct((N, C), jnp.float32),
    )(q, y2, dps, gb2)


def kernel(x, edge_index, ln1_g, ln1_b, Ws1, Wb1, bb1, gb1,
           ln2_g, ln2_b, Ws2, Wb2, bb2, gb2):
    src = edge_index[0]
    dst = edge_index[1]
    npads = E_PAD - E
    pad_i = jnp.arange(npads, dtype=jnp.int32)
    srcp = jnp.concatenate([src, pad_i % N]).reshape(E_PAD // W, W)
    dstp = jnp.concatenate([dst, N + pad_i % (NPAD - N)]).reshape(E_PAD // W, W)

    zeros_deg = jnp.zeros((ROWS_T,), jnp.float32)
    ones_w = jnp.ones((W,), jnp.float32)
    dp = _sc_degree(dstp, zeros_deg, ones_w)
    dps = dp[:, :N, None]

    Wsg1 = Ws1.reshape(D, G, H).transpose(1, 0, 2)
    y1 = _tc_fkan1(x, dps, ln1_g[None], ln1_b[None], Wsg1, Wb1, bb1[None])

    zeros_h = jnp.zeros((ROWS_T, H // 2), jnp.float32)
    p = _sc_scatter(y1, srcp, dstp, zeros_h, H // 2, True)

    D2 = D + H
    Wsg2 = Ws2.reshape(D2, G, C).transpose(1, 0, 2)
    y2 = _tc_fkan2(x, p[:, :N], y1[:, :N], dps, gb1[None], ln2_g[None],
                   ln2_b[None], Wsg2, Wb2, bb2[None])

    zeros_c = jnp.zeros((ROWS_T, C), jnp.float32)
    q = _sc_scatter(y2, srcp, dstp, zeros_c, C, False)

    return _tc_final(q[:, :N], y2, dps, gb2[None])


# speedup vs baseline: 25.6342x; 1.1334x over previous
import functools

import jax
import jax.numpy as jnp
import numpy as np
from jax import lax
from jax.experimental import pallas as pl
from jax.experimental.pallas import tpu as pltpu
from jax.experimental.pallas import tpu_sc as plsc

N = 10000
D = 128
E = 320000
H = 128
C = 40
G = 4

NTILE = 16
NCORE = 2
W = 128
NWORK = NCORE * NTILE
E_PAD = 327680
NW = E_PAD // W // NWORK
NPAD = 10240
ROWS_T = NPAD // NTILE
DEG_K = 10
NCH = 4
CH = NW // NCH

GRID = tuple(np.linspace(-2.0, 2.0, G).tolist())
INV = (G - 1) / 4.0


def _vmesh():
    return plsc.VectorSubcoreMesh(core_axis_name="c", subcore_axis_name="s")


def _sc_degree(ei3, zeros_t, ones_w):
    @functools.partial(
        pl.kernel,
        out_type=jax.ShapeDtypeStruct((NCORE, NPAD), jnp.float32),
        mesh=_vmesh(),
        scratch_types=[
            pltpu.VMEM((NCH, CH, W), jnp.int32),
            pltpu.VMEM((W,), jnp.float32),
            pltpu.VMEM_SHARED((NPAD,), jnp.float32),
            pltpu.SemaphoreType.DMA,
        ],
    )
    def k(ei_hbm, z_hbm, one_hbm, out_hbm, didx, ones_v, acc, sem):
        c = lax.axis_index("c")
        s = lax.axis_index("s")
        t0 = s * ROWS_T
        wid = c * NTILE + s
        pltpu.sync_copy(ei_hbm.at[1].at[wid], didx)
        pltpu.sync_copy(z_hbm, acc.at[pl.ds(t0, ROWS_T)])
        pltpu.sync_copy(one_hbm, ones_v)
        plsc.subcore_barrier()

        @pl.loop(0, NCH)
        def _(ch):
            @pl.loop(0, CH // DEG_K)
            def _(kk):
                descs = [
                    pltpu.async_copy(
                        ones_v, acc.at[didx.at[ch].at[kk * DEG_K + j]],
                        sem, add=True)
                    for j in range(DEG_K)
                ]
                for dsc in descs:
                    dsc.wait()

        plsc.subcore_barrier()
        pltpu.sync_copy(acc.at[pl.ds(t0, ROWS_T)],
                        out_hbm.at[c].at[pl.ds(t0, ROWS_T)])

    return k(ei3, zeros_t, ones_w)


def _sc_scatter(y, ei3, zeros_t, width, flag):
    @functools.partial(
        pl.kernel,
        out_type=jax.ShapeDtypeStruct((NCORE, NPAD, width), jnp.float32),
        mesh=_vmesh(),
        compiler_params=pltpu.CompilerParams(use_tc_tiling_on_sc=flag),
        scratch_types=[
            pltpu.VMEM((CH, W), jnp.int32),
            pltpu.VMEM((CH, W), jnp.int32),
            pltpu.VMEM((2, W, width), jnp.float32),
            pltpu.VMEM_SHARED((NPAD, width), jnp.float32),
            pltpu.SemaphoreType.DMA,
            pltpu.SemaphoreType.DMA,
            pltpu.SemaphoreType.DMA,
            pltpu.SemaphoreType.DMA,
        ],
    )
    def k(y_hbm, ei_hbm, z_hbm, out_hbm, sidx, didx, rows, acc,
          sg0, sg1, ss0, ss1):
        c = lax.axis_index("c")
        s = lax.axis_index("s")
        t0 = s * ROWS_T
        wid = c * NTILE + s
        pltpu.sync_copy(z_hbm, acc.at[pl.ds(t0, ROWS_T)])
        plsc.subcore_barrier()

        def g_start(w, b, sem):
            return pltpu.async_copy(y_hbm.at[sidx.at[w]], rows.at[b], sem)

        def g_wait(b, sem):
            pltpu.make_async_copy(y_hbm.at[sidx.at[0]], rows.at[b], sem).wait()

        def s_start(w, b, sem):
            return pltpu.async_copy(rows.at[b], acc.at[didx.at[w]], sem,
                                    add=True)

        @pl.loop(0, NCH)
        def _(ch):
            pltpu.sync_copy(ei_hbm.at[0].at[wid].at[ch], sidx)
            pltpu.sync_copy(ei_hbm.at[1].at[wid].at[ch], didx)
            g_start(0, 0, sg0)
            g_start(1, 1, sg1)

            @pl.loop(0, CH // 2 - 1)
            def _(kk):
                w0 = kk * 2
                g_wait(0, sg0)
                sd0 = s_start(w0, 0, ss0)
                g_wait(1, sg1)
                sd1 = s_start(w0 + 1, 1, ss1)
                sd0.wait()
                g_start(w0 + 2, 0, sg0)
                sd1.wait()
                g_start(w0 + 3, 1, sg1)

            g_wait(0, sg0)
            sd0 = s_start(CH - 2, 0, ss0)
            g_wait(1, sg1)
            sd1 = s_start(CH - 1, 1, ss1)
            sd0.wait()
            sd1.wait()

        plsc.subcore_barrier()
        pltpu.sync_copy(acc.at[pl.ds(t0, ROWS_T)],
                        out_hbm.at[c].at[pl.ds(t0, ROWS_T)])

    return k(y, ei3, zeros_t)


BR = 400


def _dis_block(dps):
    return lax.rsqrt(dps[0] + dps[1] + 1.0)


def _fkan_body(xin, g, b, Wsg, Wb, bb):
    m = jnp.mean(xin, axis=1, keepdims=True)
    xc = xin - m
    v = jnp.mean(xc * xc, axis=1, keepdims=True)
    h = xc * lax.rsqrt(v + 1e-5) * g + b
    acc = jnp.dot(xin * jax.nn.sigmoid(xin), Wb,
                  preferred_element_type=jnp.float32)
    for gg in range(G):
        basis = jnp.exp(-(((h - GRID[gg]) * INV) ** 2))
        acc = acc + jnp.dot(basis, Wsg[gg], preferred_element_type=jnp.float32)
    return acc + bb


def _tc_fkan1(x, dps, g, b, Wsg, Wb, bb):
    def body(x_ref, dps_ref, g_ref, b_ref, Wsg_ref, Wb_ref, bb_ref, o_ref):
        dis = _dis_block(dps_ref)
        o_ref[...] = dis * _fkan_body(x_ref[...], g_ref[...], b_ref[...],
                                      Wsg_ref, Wb_ref[...], bb_ref[...])

    return pl.pallas_call(
        body,
        grid=(N // BR,),
        in_specs=[
            pl.BlockSpec((BR, D), lambda i: (i, 0)),
            pl.BlockSpec((2, BR, 1), lambda i: (0, i, 0)),
            pl.BlockSpec((1, D), lambda i: (0, 0)),
            pl.BlockSpec((1, D), lambda i: (0, 0)),
            pl.BlockSpec((G, D, H), lambda i: (0, 0, 0)),
            pl.BlockSpec((D, H), lambda i: (0, 0)),
            pl.BlockSpec((1, H), lambda i: (0, 0)),
        ],
        out_specs=pl.BlockSpec((BR, H), lambda i: (i, 0)),
        out_shape=jax.ShapeDtypeStruct((N, H), jnp.float32),
    )(x, dps, g, b, Wsg, Wb, bb)


def _tc_fkan2(x, p, y1, dps, gb1, g2, b2, Wsg2, Wb2, bb2):
    D2 = D + H

    def body(x_ref, p_ref, y1_ref, dps_ref, gb1_ref, g_ref, b_ref, Wsg_ref,
             Wb_ref, bb_ref, o_ref):
        dis = _dis_block(dps_ref)
        h1 = dis * (p_ref[0] + p_ref[1] + y1_ref[...]) + gb1_ref[...]
        hcat = jnp.concatenate([x_ref[...], h1], axis=1)
        y = _fkan_body(hcat, g_ref[...], b_ref[...], Wsg_ref, Wb_ref[...],
                       bb_ref[...])
        o_ref[...] = dis * y

    return pl.pallas_call(
        body,
        grid=(N // BR,),
        in_specs=[
            pl.BlockSpec((BR, D), lambda i: (i, 0)),
            pl.BlockSpec((2, BR, H), lambda i: (0, i, 0)),
            pl.BlockSpec((BR, H), lambda i: (i, 0)),
            pl.BlockSpec((2, BR, 1), lambda i: (0, i, 0)),
            pl.BlockSpec((1, H), lambda i: (0, 0)),
            pl.BlockSpec((1, D2), lambda i: (0, 0)),
            pl.BlockSpec((1, D2), lambda i: (0, 0)),
            pl.BlockSpec((G, D2, C), lambda i: (0, 0, 0)),
            pl.BlockSpec((D2, C), lambda i: (0, 0)),
            pl.BlockSpec((1, C), lambda i: (0, 0)),
        ],
        out_specs=pl.BlockSpec((BR, C), lambda i: (i, 0)),
        out_shape=jax.ShapeDtypeStruct((N, C), jnp.float32),
    )(x, p, y1, dps, gb1, g2, b2, Wsg2, Wb2, bb2)


def _tc_final(q, y2, dps, gb2):
    def body(q_ref, y2_ref, dps_ref, gb2_ref, o_ref):
        dis = _dis_block(dps_ref)
        o_ref[...] = dis * (q_ref[0] + q_ref[1] + y2_ref[...]) + gb2_ref[...]

    return pl.pallas_call(
        body,
        grid=(N // BR,),
        in_specs=[
            pl.BlockSpec((2, BR, C), lambda i: (0, i, 0)),
            pl.BlockSpec((BR, C), lambda i: (i, 0)),
            pl.BlockSpec((2, BR, 1), lambda i: (0, i, 0)),
            pl.BlockSpec((1, C), lambda i: (0, 0)),
        ],
        out_specs=pl.BlockSpec((BR, C), lambda i: (i, 0)),
        out_shape=jax.ShapeDtypeStruct((N, C), jnp.float32),
    )(q, y2, dps, gb2)


def kernel(x, edge_index, ln1_g, ln1_b, Ws1, Wb1, bb1, gb1,
           ln2_g, ln2_b, Ws2, Wb2, bb2, gb2):
    pad_i = jnp.arange(E_PAD - E, dtype=jnp.int32)
    pad2 = jnp.stack([pad_i % N, N + pad_i % (NPAD - N)])
    ei3 = jnp.concatenate([edge_index, pad2], axis=1).reshape(
        2, NWORK, NCH, CH, W)

    zeros_deg = jnp.zeros((ROWS_T,), jnp.float32)
    ones_w = jnp.ones((W,), jnp.float32)
    dps = _sc_degree(ei3, zeros_deg, ones_w)[..., None]

    Wsg1 = Ws1.reshape(D, G, H).transpose(1, 0, 2)
    y1 = _tc_fkan1(x, dps, ln1_g[None], ln1_b[None], Wsg1, Wb1, bb1[None])

    zeros_h = jnp.zeros((ROWS_T, H), jnp.float32)
    p = _sc_scatter(y1, ei3, zeros_h, H, True)

    D2 = D + H
    Wsg2 = Ws2.reshape(D2, G, C).transpose(1, 0, 2)
    y2 = _tc_fkan2(x, p, y1, dps, gb1[None], ln2_g[None], ln2_b[None],
                   Wsg2, Wb2, bb2[None])

    zeros_c = jnp.zeros((ROWS_T, C), jnp.float32)
    q = _sc_scatter(y2, ei3, zeros_c, C, False)

    return _tc_final(q, y2, dps, gb2[None])


# speedup vs baseline: 29.5735x; 1.1537x over previous
import functools

import jax
import jax.numpy as jnp
import numpy as np
from jax import lax
from jax.experimental import pallas as pl
from jax.experimental.pallas import tpu as pltpu
from jax.experimental.pallas import tpu_sc as plsc

N = 10000
D = 128
E = 320000
H = 128
C = 40
G = 4

NTILE = 16
NCORE = 2
W = 128
NWORK = NCORE * NTILE
E_PAD = 327680
NW = E_PAD // W // NWORK
NPAD = 10240
ROWS_T = NPAD // NTILE
DEG_K = 10
NCH = 4
CH = NW // NCH
NCH_MAIN = E // W // CH
NPADW = (E_PAD - E) // W

GRID = tuple(np.linspace(-2.0, 2.0, G).tolist())
INV = (G - 1) / 4.0


def _vmesh():
    return plsc.VectorSubcoreMesh(core_axis_name="c", subcore_axis_name="s")


def _load_chunk(eim, eip, cg, bufs):
    @pl.when(cg < NCH_MAIN)
    def _():
        for row, buf in bufs:
            pltpu.sync_copy(eim.at[row].at[cg], buf)

    @pl.when(cg >= NCH_MAIN)
    def _():
        for row, buf in bufs:
            pltpu.sync_copy(eip.at[row].at[cg - NCH_MAIN], buf)


def _sc_degree(eim, eip, zeros_t, ones_w):
    @functools.partial(
        pl.kernel,
        out_type=jax.ShapeDtypeStruct((NCORE, NPAD), jnp.float32),
        mesh=_vmesh(),
        scratch_types=[
            pltpu.VMEM((CH, W), jnp.int32),
            pltpu.VMEM((W,), jnp.float32),
            pltpu.VMEM_SHARED((NPAD,), jnp.float32),
            pltpu.SemaphoreType.DMA,
        ],
    )
    def k(eim_hbm, eip_hbm, z_hbm, one_hbm, out_hbm, didx, ones_v, acc, sem):
        c = lax.axis_index("c")
        s = lax.axis_index("s")
        t0 = s * ROWS_T
        wid = c * NTILE + s
        pltpu.sync_copy(z_hbm, acc.at[pl.ds(t0, ROWS_T)])
        pltpu.sync_copy(one_hbm, ones_v)
        plsc.subcore_barrier()

        @pl.loop(0, NCH)
        def _(ch):
            _load_chunk(eim_hbm, eip_hbm, wid * NCH + ch, [(1, didx)])

            @pl.loop(0, CH // DEG_K)
            def _(kk):
                descs = [
                    pltpu.async_copy(ones_v,
                                     acc.at[didx.at[kk * DEG_K + j]],
                                     sem, add=True)
                    for j in range(DEG_K)
                ]
                for dsc in descs:
                    dsc.wait()

        plsc.subcore_barrier()
        pltpu.sync_copy(acc.at[pl.ds(t0, ROWS_T)],
                        out_hbm.at[c].at[pl.ds(t0, ROWS_T)])

    return k(eim, eip, zeros_t, ones_w)


def _sc_scatter(y, eim, eip, zeros_t, width, flag, nbuf):
    @functools.partial(
        pl.kernel,
        out_type=jax.ShapeDtypeStruct((NCORE, NPAD, width), jnp.float32),
        mesh=_vmesh(),
        compiler_params=pltpu.CompilerParams(use_tc_tiling_on_sc=flag),
        scratch_types=[
            pltpu.VMEM((CH, W), jnp.int32),
            pltpu.VMEM((CH, W), jnp.int32),
            pltpu.VMEM((nbuf, W, width), jnp.float32),
            pltpu.VMEM_SHARED((NPAD, width), jnp.float32),
        ] + [pltpu.SemaphoreType.DMA] * (2 * nbuf),
    )
    def k(y_hbm, eim_hbm, eip_hbm, z_hbm, out_hbm, sidx, didx, rows, acc,
          *sems):
        sg = sems[:nbuf]
        ss = sems[nbuf:]
        c = lax.axis_index("c")
        s = lax.axis_index("s")
        t0 = s * ROWS_T
        wid = c * NTILE + s
        pltpu.sync_copy(z_hbm, acc.at[pl.ds(t0, ROWS_T)])
        plsc.subcore_barrier()

        def g_start(w, b):
            return pltpu.async_copy(y_hbm.at[sidx.at[w]], rows.at[b], sg[b])

        def g_wait(b):
            pltpu.make_async_copy(y_hbm.at[sidx.at[0]], rows.at[b],
                                  sg[b]).wait()

        def s_start(w, b):
            return pltpu.async_copy(rows.at[b], acc.at[didx.at[w]], ss[b],
                                    add=True)

        @pl.loop(0, NCH)
        def _(ch):
            _load_chunk(eim_hbm, eip_hbm, wid * NCH + ch,
                        [(0, sidx), (1, didx)])
            for j in range(nbuf):
                g_start(j, j)

            @pl.loop(0, CH // nbuf - 1)
            def _(qq):
                w0 = qq * nbuf
                sds = []
                for j in range(nbuf):
                    g_wait(j)
                    sds.append(s_start(w0 + j, j))
                for j in range(nbuf):
                    sds[j].wait()
                    g_start(w0 + nbuf + j, j)

            w0 = CH - nbuf
            sds = []
            for j in range(nbuf):
                g_wait(j)
                sds.append(s_start(w0 + j, j))
            for j in range(nbuf):
                sds[j].wait()

        plsc.subcore_barrier()
        pltpu.sync_copy(acc.at[pl.ds(t0, ROWS_T)],
                        out_hbm.at[c].at[pl.ds(t0, ROWS_T)])

    return k(y, eim, eip, zeros_t)


BR = 512
NBLK = NPAD // BR
BRF = 2048
NBLKF = NPAD // BRF


def _fkan_cat(xin, g, b, Wcat, bb):
    m = jnp.mean(xin, axis=1, keepdims=True)
    xc = xin - m
    v = jnp.mean(xc * xc, axis=1, keepdims=True)
    h = xc * lax.rsqrt(v + 1e-5) * g + b
    parts = [jnp.exp(-(((h - GRID[gg]) * INV) ** 2)) for gg in range(G)]
    parts.append(xin * jax.nn.sigmoid(xin))
    bcat = jnp.concatenate(parts, axis=1)
    return jnp.dot(bcat, Wcat, preferred_element_type=jnp.float32) + bb


def _tc_fkan1(x, dp, g, b, Wcat, bb):
    def body(x_ref, dp_ref, g_ref, b_ref, Wcat_ref, bb_ref, o_ref, db_ref):
        dsum = dp_ref[0] + dp_ref[1] + 1.0
        disb = jnp.broadcast_to(lax.rsqrt(dsum)[:, None], (BR, D))
        db_ref[...] = disb
        o_ref[...] = disb * _fkan_cat(x_ref[...], g_ref[...], b_ref[...],
                                      Wcat_ref[...], bb_ref[...])

    return pl.pallas_call(
        body,
        grid=(NBLK,),
        in_specs=[
            pl.BlockSpec((BR, D), lambda i: (i, 0)),
            pl.BlockSpec((2, BR), lambda i: (0, i)),
            pl.BlockSpec((1, D), lambda i: (0, 0)),
            pl.BlockSpec((1, D), lambda i: (0, 0)),
            pl.BlockSpec(((G + 1) * D, H), lambda i: (0, 0)),
            pl.BlockSpec((1, H), lambda i: (0, 0)),
        ],
        out_specs=[
            pl.BlockSpec((BR, H), lambda i: (i, 0)),
            pl.BlockSpec((BR, D), lambda i: (i, 0)),
        ],
        out_shape=[
            jax.ShapeDtypeStruct((N, H), jnp.float32),
            jax.ShapeDtypeStruct((N, D), jnp.float32),
        ],
    )(x, dp, g, b, Wcat, bb)


def _tc_fkan2(x, p, y1, disb, gb1, g2, b2, Wcat2, bb2):
    D2 = D + H

    def body(x_ref, p_ref, y1_ref, db_ref, gb1_ref, g_ref, b_ref, Wcat_ref,
             bb_ref, o_ref):
        disb = db_ref[...]
        h1 = disb * (p_ref[0] + p_ref[1] + y1_ref[...]) + gb1_ref[...]
        hcat = jnp.concatenate([x_ref[...], h1], axis=1)
        y = _fkan_cat(hcat, g_ref[...], b_ref[...], Wcat_ref[...], bb_ref[...])
        o_ref[...] = disb[:, :C] * y

    return pl.pallas_call(
        body,
        grid=(NBLK,),
        in_specs=[
            pl.BlockSpec((BR, D), lambda i: (i, 0)),
            pl.BlockSpec((2, BR, H), lambda i: (0, i, 0)),
            pl.BlockSpec((BR, H), lambda i: (i, 0)),
            pl.BlockSpec((BR, D), lambda i: (i, 0)),
            pl.BlockSpec((1, H), lambda i: (0, 0)),
            pl.BlockSpec((1, D2), lambda i: (0, 0)),
            pl.BlockSpec((1, D2), lambda i: (0, 0)),
            pl.BlockSpec(((G + 1) * D2, C), lambda i: (0, 0)),
            pl.BlockSpec((1, C), lambda i: (0, 0)),
        ],
        out_specs=pl.BlockSpec((BR, C), lambda i: (i, 0)),
        out_shape=jax.ShapeDtypeStruct((N, C), jnp.float32),
    )(x, p, y1, disb, gb1, g2, b2, Wcat2, bb2)


def _tc_final(q, y2, disb, gb2):
    def body(q_ref, y2_ref, db_ref, gb2_ref, o_ref):
        o_ref[...] = (db_ref[:, :C] * (q_ref[0] + q_ref[1] + y2_ref[...])
                      + gb2_ref[...])

    return pl.pallas_call(
        body,
        grid=(NBLKF,),
        in_specs=[
            pl.BlockSpec((2, BRF, C), lambda i: (0, i, 0)),
            pl.BlockSpec((BRF, C), lambda i: (i, 0)),
            pl.BlockSpec((BRF, D), lambda i: (i, 0)),
            pl.BlockSpec((1, C), lambda i: (0, 0)),
        ],
        out_specs=pl.BlockSpec((BRF, C), lambda i: (i, 0)),
        out_shape=jax.ShapeDtypeStruct((N, C), jnp.float32),
    )(q, y2, disb, gb2)


def kernel(x, edge_index, ln1_g, ln1_b, Ws1, Wb1, bb1, gb1,
           ln2_g, ln2_b, Ws2, Wb2, bb2, gb2):
    eim = edge_index.reshape(2, NCH_MAIN, CH, W)
    pad_i = jnp.arange(E_PAD - E, dtype=jnp.int32)
    eip = jnp.stack([pad_i % N, N + pad_i % (NPAD - N)]).reshape(
        2, NPADW // CH, CH, W)

    zeros_deg = jnp.zeros((ROWS_T,), jnp.float32)
    ones_w = jnp.ones((W,), jnp.float32)
    dp = _sc_degree(eim, eip, zeros_deg, ones_w)

    Wcat1 = jnp.concatenate(
        [Ws1.reshape(D, G, H).transpose(1, 0, 2).reshape(G * D, H), Wb1], 0)
    y1, disb = _tc_fkan1(x, dp, ln1_g[None], ln1_b[None], Wcat1, bb1[None])

    zeros_h = jnp.zeros((ROWS_T, H), jnp.float32)
    p = _sc_scatter(y1, eim, eip, zeros_h, H, True, 2)

    D2 = D + H
    Wcat2 = jnp.concatenate(
        [Ws2.reshape(D2, G, C).transpose(1, 0, 2).reshape(G * D2, C), Wb2], 0)
    y2 = _tc_fkan2(x, p, y1, disb, gb1[None], ln2_g[None], ln2_b[None],
                   Wcat2, bb2[None])

    zeros_c = jnp.zeros((ROWS_T, C), jnp.float32)
    q = _sc_scatter(y2, eim, eip, zeros_c, C, False, 4)

    return _tc_final(q, y2, disb, gb2[None])
